# baseline (speedup 1.0000x reference)
import functools

import jax
import jax.numpy as jnp
import numpy as np
from jax import lax
from jax.experimental import pallas as pl
from jax.experimental.pallas import tpu as pltpu
from jax.experimental.pallas import tpu_sc as plsc

D = 64
H = 1024
BN_EPS = 1e-5


def _make_gather(B, n_users, n_items):
    info = plsc.get_sparse_core_info()
    nc, ns = info.num_cores, info.num_subcores
    nw = nc * ns
    assert B % (8 * nw) == 0
    b_per_w = B // nw
    mesh = plsc.VectorSubcoreMesh(core_axis_name="c", subcore_axis_name="s")

    @functools.partial(
        pl.kernel,
        mesh=mesh,
        compiler_params=pltpu.CompilerParams(use_tc_tiling_on_sc=False),
        out_type=[
            jax.ShapeDtypeStruct((B, D), jnp.float32),
            jax.ShapeDtypeStruct((B, D), jnp.float32),
        ],
        scratch_types=[
            pltpu.VMEM((b_per_w,), jnp.int32),
            pltpu.VMEM((b_per_w,), jnp.int32),
            pltpu.VMEM((b_per_w, D), jnp.float32),
            pltpu.VMEM((b_per_w, D), jnp.float32),
            pltpu.SemaphoreType.DMA,
            pltpu.SemaphoreType.DMA,
        ],
    )
    def gather_k(u_hbm, m_hbm, uemb_hbm, memb_hbm, ue_out, me_out,
                 uidx_v, midx_v, urows_v, mrows_v, sem_u, sem_m):
        wid = lax.axis_index("s") * nc + lax.axis_index("c")
        base = wid * b_per_w
        pltpu.sync_copy(u_hbm.at[pl.ds(base, b_per_w)], uidx_v)
        pltpu.sync_copy(m_hbm.at[pl.ds(base, b_per_w)], midx_v)
        cp_u = pltpu.async_copy(uemb_hbm.at[uidx_v], urows_v, sem_u)
        cp_m = pltpu.async_copy(memb_hbm.at[midx_v], mrows_v, sem_m)
        cp_u.wait()
        cp_m.wait()
        pltpu.sync_copy(urows_v, ue_out.at[pl.ds(base, b_per_w)])
        pltpu.sync_copy(mrows_v, me_out.at[pl.ds(base, b_per_w)])

    return gather_k


def _mlp_body(ue_ref, me_ref, w1u_ref, w1m_ref, b1_ref, gamma_ref, beta_ref,
              w2t_ref, b2_ref, out_ref):
    h = jnp.dot(ue_ref[...], w1u_ref[...], preferred_element_type=jnp.float32)
    h = h + jnp.dot(me_ref[...], w1m_ref[...],
                    preferred_element_type=jnp.float32)
    h = h + b1_ref[...]
    h = jnp.maximum(h, 0.0)
    inv = np.float32(1.0 / np.sqrt(1.0 + BN_EPS))
    h = h * (gamma_ref[...] * inv) + beta_ref[...]
    out_ref[...] = jnp.sum(h * w2t_ref[...], axis=1, keepdims=True) + b2_ref[...]


def _make_mlp(B, bsz):
    grid = (B // bsz,)
    return pl.pallas_call(
        _mlp_body,
        grid=grid,
        in_specs=[
            pl.BlockSpec((bsz, D), lambda i: (i, 0)),
            pl.BlockSpec((bsz, D), lambda i: (i, 0)),
            pl.BlockSpec((D, H), lambda i: (0, 0)),
            pl.BlockSpec((D, H), lambda i: (0, 0)),
            pl.BlockSpec((1, H), lambda i: (0, 0)),
            pl.BlockSpec((1, H), lambda i: (0, 0)),
            pl.BlockSpec((1, H), lambda i: (0, 0)),
            pl.BlockSpec((1, H), lambda i: (0, 0)),
            pl.BlockSpec((1, 1), lambda i: (0, 0)),
        ],
        out_specs=pl.BlockSpec((bsz, 1), lambda i: (i, 0)),
        out_shape=jax.ShapeDtypeStruct((B, 1), jnp.float32),
    )


def kernel(u, m, u_emb, m_emb, W1, b1, gamma, beta, W2, b2):
    B = u.shape[0]
    gather_k = _make_gather(B, u_emb.shape[0], m_emb.shape[0])
    ue, me = gather_k(u, m, u_emb, m_emb)
    mlp = _make_mlp(B, 2048)
    return mlp(
        ue, me, W1[:D], W1[D:],
        b1.reshape(1, H), gamma.reshape(1, H), beta.reshape(1, H),
        W2.reshape(1, H), b2.reshape(1, 1),
    )
